# baseline (speedup 1.0000x reference)
import functools
import jax
import jax.numpy as jnp
from jax import lax
from jax.experimental import pallas as pl
from jax.experimental.pallas import tpu as pltpu
from jax.experimental.pallas import tpu_sc as plsc

N = 10000
E = 160000
FH = 128
NT = 16
BSZ = 32
NB = 324
EPT = NB * BSZ
E_PAD = NT * EPT
BN = 1000


def _sd_epilogue(h, as_ref, ad_ref, sd_ref, cm_ref, acc):
    s = jnp.sum(h * as_ref[...], axis=1)
    d = jnp.sum(h * ad_ref[...], axis=1)
    sd_ref[...] = jnp.concatenate(
        [s[:, None], d[:, None], jnp.zeros((s.shape[0], 6), jnp.float32)], axis=1)
    i = pl.program_id(0)
    bs, bd = jnp.max(s), jnp.max(d)

    @pl.when(i == 0)
    def _():
        acc[0] = bs
        acc[1] = bd

    @pl.when(i > 0)
    def _():
        acc[0] = jnp.maximum(acc[0], bs)
        acc[1] = jnp.maximum(acc[1], bd)

    @pl.when(i == pl.num_programs(0) - 1)
    def _():
        cm_ref[...] = jnp.full((8, 128), jnp.maximum(acc[0] + acc[1], 0.0),
                               jnp.float32)


def _tc_proj_body(x_ref, w_ref, as_ref, ad_ref, h_ref, sd_ref, cm_ref, acc):
    h = jnp.dot(x_ref[...], w_ref[...], preferred_element_type=jnp.float32)
    h_ref[0] = h[:, :FH]
    h_ref[1] = h[:, FH:]
    _sd_epilogue(h, as_ref, ad_ref, sd_ref, cm_ref, acc)


def _tc_proj(x, w, a_src, a_dst):
    fin = x.shape[1]
    return pl.pallas_call(
        _tc_proj_body,
        grid=(N // BN,),
        in_specs=[
            pl.BlockSpec((BN, fin), lambda i: (i, 0)),
            pl.BlockSpec((fin, 256), lambda i: (0, 0)),
            pl.BlockSpec((1, 256), lambda i: (0, 0)),
            pl.BlockSpec((1, 256), lambda i: (0, 0)),
        ],
        out_specs=[
            pl.BlockSpec((2, BN, FH), lambda i: (0, i, 0)),
            pl.BlockSpec((BN, 8), lambda i: (i, 0)),
            pl.BlockSpec((8, 128), lambda i: (0, 0)),
        ],
        out_shape=[
            jax.ShapeDtypeStruct((2, N, FH), jnp.float32),
            jax.ShapeDtypeStruct((N, 8), jnp.float32),
            jax.ShapeDtypeStruct((8, 128), jnp.float32),
        ],
        scratch_shapes=[pltpu.SMEM((2,), jnp.float32)],
    )(x, w, a_src[None, :], a_dst[None, :])


def _tc_epi_proj_body(yprev_ref, agg_ref, den_ref, b_ref, w_ref, as_ref,
                      ad_ref, y_ref, h_ref, sd_ref, cm_ref, acc):
    agg = jnp.concatenate([agg_ref[0], agg_ref[1]], axis=1)
    agg = agg * (1.0 / (den_ref[...] + 1e-16))
    y = jnp.maximum(agg + b_ref[...], 0.0)
    if yprev_ref is not None:
        y = y + yprev_ref[...]
    y_ref[...] = y
    h = jnp.dot(y, w_ref[...], preferred_element_type=jnp.float32)
    h_ref[0] = h[:, :FH]
    h_ref[1] = h[:, FH:]
    _sd_epilogue(h, as_ref, ad_ref, sd_ref, cm_ref, acc)


def _tc_epi_proj(yprev, agg, den, b, w, a_src, a_dst):
    if yprev is not None:
        body = _tc_epi_proj_body
        args = (yprev, agg, den, b[None, :], w, a_src[None, :], a_dst[None, :])
        prev_specs = [pl.BlockSpec((BN, 256), lambda i: (i, 0))]
    else:
        body = functools.partial(_tc_epi_proj_body, None)
        args = (agg, den, b[None, :], w, a_src[None, :], a_dst[None, :])
        prev_specs = []
    return pl.pallas_call(
        body,
        grid=(N // BN,),
        in_specs=prev_specs + [
            pl.BlockSpec((2, BN, FH), lambda i: (0, i, 0)),
            pl.BlockSpec((BN, 1), lambda i: (i, 0)),
            pl.BlockSpec((1, 256), lambda i: (0, 0)),
            pl.BlockSpec((256, 256), lambda i: (0, 0)),
            pl.BlockSpec((1, 256), lambda i: (0, 0)),
            pl.BlockSpec((1, 256), lambda i: (0, 0)),
        ],
        out_specs=[
            pl.BlockSpec((BN, 256), lambda i: (i, 0)),
            pl.BlockSpec((2, BN, FH), lambda i: (0, i, 0)),
            pl.BlockSpec((BN, 8), lambda i: (i, 0)),
            pl.BlockSpec((8, 128), lambda i: (0, 0)),
        ],
        out_shape=[
            jax.ShapeDtypeStruct((N, 256), jnp.float32),
            jax.ShapeDtypeStruct((2, N, FH), jnp.float32),
            jax.ShapeDtypeStruct((N, 8), jnp.float32),
            jax.ShapeDtypeStruct((8, 128), jnp.float32),
        ],
        scratch_shapes=[pltpu.SMEM((2,), jnp.float32)],
    )(*args)


def _tc_final_body(yprev_ref, agg_ref, den_ref, b_ref, out_ref):
    agg = jnp.concatenate([agg_ref[0], agg_ref[1]], axis=1)
    agg = agg * (1.0 / (den_ref[...] + 1e-16))
    out_ref[...] = yprev_ref[...] + jnp.maximum(agg + b_ref[...], 0.0)


def _tc_final(yprev, agg, den, b):
    return pl.pallas_call(
        _tc_final_body,
        grid=(N // BN,),
        in_specs=[
            pl.BlockSpec((BN, 256), lambda i: (i, 0)),
            pl.BlockSpec((2, BN, FH), lambda i: (0, i, 0)),
            pl.BlockSpec((BN, 1), lambda i: (i, 0)),
            pl.BlockSpec((1, 256), lambda i: (0, 0)),
        ],
        out_specs=pl.BlockSpec((BN, 256), lambda i: (i, 0)),
        out_shape=jax.ShapeDtypeStruct((N, 256), jnp.float32),
    )(yprev, agg, den, b[None, :])


def _sc_edge_body(h_hbm, s_hbm, d_hbm, c_hbm, src_hbm, dst_hbm,
                  acc_hbm, den_hbm,
                  srcb, srcoff, dstb, sv, dv, pb, rowbuf, sbuf, zb, stg, cloc,
                  s_sh, d_sh, denom_sh, acc_sh,
                  sem_r0, sem_r1, sem_r2, sem_s0, sem_s1, sem_s2,
                  sem_d0, sem_d1, sem_a0, sem_a1):
    c = lax.axis_index("c")
    sid = lax.axis_index("s")
    zerov = jnp.zeros((16,), jnp.float32)
    sem_row = (sem_r0, sem_r1, sem_r2)
    sem_sd = (sem_s0, sem_s1, sem_s2)
    sem_den = (sem_d0, sem_d1)
    sem_acc = (sem_a0, sem_a1)

    pltpu.sync_copy(c_hbm, cloc)
    C = cloc[pl.ds(0, 16)][0]

    for i in range(40):
        zb[pl.ds(i * 16, 16)] = zerov

    def _zrow(g, _):
        for ii in range(16):
            for k in range(8):
                rowbuf[0, g * 16 + ii, pl.ds(k * 16, 16)] = zerov
        return ()
    lax.fori_loop(0, BSZ // 16, _zrow, ())

    rbase = sid * 632

    @pl.when(sid < 15)
    def _():
        for i in range(19):
            pltpu.sync_copy(rowbuf.at[0],
                            acc_sh.at[pl.ds(rbase + i * 32, 32)])
        pltpu.sync_copy(rowbuf.at[0, pl.ds(0, 24)],
                        acc_sh.at[pl.ds(rbase + 608, 24)])
        pltpu.sync_copy(zb.at[pl.ds(0, 632)], denom_sh.at[pl.ds(rbase, 632)])
        pltpu.sync_copy(s_hbm.at[pl.ds(rbase, 632)], stg.at[pl.ds(0, 632)])
        pltpu.sync_copy(stg.at[pl.ds(0, 632)], s_sh.at[pl.ds(rbase, 632)])
        pltpu.sync_copy(d_hbm.at[pl.ds(rbase, 632)], stg.at[pl.ds(0, 632)])
        pltpu.sync_copy(stg.at[pl.ds(0, 632)], d_sh.at[pl.ds(rbase, 632)])

    @pl.when(sid == 15)
    def _():
        for i in range(16):
            pltpu.sync_copy(rowbuf.at[0],
                            acc_sh.at[pl.ds(rbase + i * 32, 32)])
        pltpu.sync_copy(rowbuf.at[0, pl.ds(0, 8)],
                        acc_sh.at[pl.ds(rbase + 512, 8)])
        pltpu.sync_copy(zb.at[pl.ds(0, 520)], denom_sh.at[pl.ds(rbase, 520)])
        pltpu.sync_copy(s_hbm.at[pl.ds(rbase, 520)], stg.at[pl.ds(0, 520)])
        pltpu.sync_copy(stg.at[pl.ds(0, 520)], s_sh.at[pl.ds(rbase, 520)])
        pltpu.sync_copy(d_hbm.at[pl.ds(rbase, 520)], stg.at[pl.ds(0, 520)])
        pltpu.sync_copy(stg.at[pl.ds(0, 520)], d_sh.at[pl.ds(rbase, 520)])

    plsc.subcore_barrier()

    off = c * N
    ebase = sid * EPT

    def fire(m, rs, d6):
        pltpu.sync_copy(src_hbm.at[sid, m], srcb.at[rs])
        pltpu.sync_copy(dst_hbm.at[sid, m], dstb.at[d6])
        for k in range(BSZ // 16):
            srcoff[rs, pl.ds(k * 16, 16)] = (
                srcb[rs, pl.ds(k * 16, 16)] + off)
        pltpu.async_copy(h_hbm.at[srcoff.at[rs]], rowbuf.at[rs], sem_row[rs])
        pltpu.async_copy(s_sh.at[srcb.at[rs]], sv.at[rs], sem_sd[rs])
        pltpu.async_copy(d_sh.at[dstb.at[d6]], dv.at[rs], sem_sd[rs])

    def proc(m, rs, d6, ps):
        @pl.when(m >= 2)
        def _():
            pltpu.make_async_copy(
                h_hbm.at[pl.ds(0, BSZ)], sbuf.at[ps], sem_acc[ps]).wait()
            pltpu.make_async_copy(
                s_hbm.at[pl.ds(0, BSZ)], pb.at[ps], sem_den[ps]).wait()

        pltpu.make_async_copy(
            s_hbm.at[pl.ds(0, BSZ)], sv.at[rs], sem_sd[rs]).wait()
        pltpu.make_async_copy(
            s_hbm.at[pl.ds(0, BSZ)], dv.at[rs], sem_sd[rs]).wait()

        for k in range(BSZ // 16):
            e = sv[rs, pl.ds(k * 16, 16)] + dv[rs, pl.ds(k * 16, 16)]
            e = jnp.where(e >= 0.0, e, 0.2 * e)
            p = jnp.exp(e - C)
            gid = ebase + m * BSZ + k * 16 + lax.iota(jnp.int32, 16)
            pb[ps, pl.ds(k * 16, 16)] = jnp.where(gid < E, p, 0.0)
        pltpu.async_copy(pb.at[ps], denom_sh.at[dstb.at[d6]], sem_den[ps],
                         add=True)

        pltpu.make_async_copy(
            h_hbm.at[pl.ds(0, BSZ)], rowbuf.at[rs], sem_row[rs]).wait()

        def _grp(g, _):
            av = pb[ps, pl.ds(g * 16, 16)]
            for ii in range(16):
                a = av[ii]
                r = g * 16 + ii
                for k in range(8):
                    sbuf[ps, r, pl.ds(k * 16, 16)] = (
                        rowbuf[rs, r, pl.ds(k * 16, 16)] * a)
            return ()
        lax.fori_loop(0, BSZ // 16, _grp, ())
        pltpu.async_copy(sbuf.at[ps], acc_sh.at[dstb.at[d6]], sem_acc[ps],
                         add=True)

    fire(0, 0, 0)
    fire(1, 1, 1)

    def _outer(jo, _):
        j = 6 * jo
        fire(j + 2, 2, 2)
        proc(j, 0, 0, 0)
        fire(j + 3, 0, 3)
        proc(j + 1, 1, 1, 1)
        fire(j + 4, 1, 4)
        proc(j + 2, 2, 2, 0)
        fire(j + 5, 2, 5)
        proc(j + 3, 0, 3, 1)

        @pl.when(j + 6 < NB)
        def _():
            fire(j + 6, 0, 0)
        proc(j + 4, 1, 4, 0)

        @pl.when(j + 7 < NB)
        def _():
            fire(j + 7, 1, 1)
        proc(j + 5, 2, 5, 1)
        return ()
    lax.fori_loop(0, NB // 6, _outer, ())

    for ps in range(2):
        pltpu.make_async_copy(
            h_hbm.at[pl.ds(0, BSZ)], sbuf.at[ps], sem_acc[ps]).wait()
        pltpu.make_async_copy(
            s_hbm.at[pl.ds(0, BSZ)], pb.at[ps], sem_den[ps]).wait()

    plsc.subcore_barrier()

    @pl.when(sid < 15)
    def _():
        pltpu.sync_copy(acc_sh.at[pl.ds(rbase, 632)],
                        acc_hbm.at[pl.ds(c * N + rbase, 632)])

        @pl.when(c == 0)
        def _():
            pltpu.sync_copy(denom_sh.at[pl.ds(rbase, 632)],
                            stg.at[pl.ds(0, 632)])
            pltpu.sync_copy(stg.at[pl.ds(0, 632)],
                            den_hbm.at[pl.ds(rbase, 632)])

    @pl.when(sid == 15)
    def _():
        pltpu.sync_copy(acc_sh.at[pl.ds(rbase, 520)],
                        acc_hbm.at[pl.ds(c * N + rbase, 520)])

        @pl.when(c == 0)
        def _():
            pltpu.sync_copy(denom_sh.at[pl.ds(rbase, 520)],
                            stg.at[pl.ds(0, 520)])
            pltpu.sync_copy(stg.at[pl.ds(0, 520)],
                            den_hbm.at[pl.ds(rbase, 520)])


_sc_edge = functools.partial(
    pl.kernel,
    mesh=plsc.VectorSubcoreMesh(core_axis_name="c", subcore_axis_name="s"),
    compiler_params=pltpu.CompilerParams(needs_layout_passes=False),
    out_type=[
        jax.ShapeDtypeStruct((2 * N, FH), jnp.float32),
        jax.ShapeDtypeStruct((N,), jnp.float32),
    ],
    scratch_types=[
        pltpu.VMEM((3, BSZ), jnp.int32),
        pltpu.VMEM((3, BSZ), jnp.int32),
        pltpu.VMEM((6, BSZ), jnp.int32),
        pltpu.VMEM((3, BSZ), jnp.float32),
        pltpu.VMEM((3, BSZ), jnp.float32),
        pltpu.VMEM((2, BSZ), jnp.float32),
        pltpu.VMEM((3, BSZ, FH), jnp.float32),
        pltpu.VMEM((2, BSZ, FH), jnp.float32),
        pltpu.VMEM((640,), jnp.float32),
        pltpu.VMEM((640,), jnp.float32),
        pltpu.VMEM((16,), jnp.float32),
        pltpu.VMEM_SHARED((N,), jnp.float32),
        pltpu.VMEM_SHARED((N,), jnp.float32),
        pltpu.VMEM_SHARED((N,), jnp.float32),
        pltpu.VMEM_SHARED((N, FH), jnp.float32),
        pltpu.SemaphoreType.DMA,
        pltpu.SemaphoreType.DMA,
        pltpu.SemaphoreType.DMA,
        pltpu.SemaphoreType.DMA,
        pltpu.SemaphoreType.DMA,
        pltpu.SemaphoreType.DMA,
        pltpu.SemaphoreType.DMA,
        pltpu.SemaphoreType.DMA,
        pltpu.SemaphoreType.DMA,
        pltpu.SemaphoreType.DMA,
    ],
)(_sc_edge_body)


def _sc_layer(h_split, sd, cm, srcp, dstp):
    h_flat = h_split.reshape(2 * N, FH)
    cvec = cm.reshape(-1)[:16]
    agg, den = _sc_edge(h_flat, sd[:, 0], sd[:, 1], cvec, srcp, dstp)
    return agg.reshape(2, N, FH), den.reshape(N, 1)


def kernel(x, edge_index, W0, a_src0, a_dst0, b0, W1, a_src1, a_dst1, b1,
           W2, a_src2, a_dst2, b2):
    pad = jnp.zeros((E_PAD - E,), jnp.int32)
    srcp = jnp.concatenate([edge_index[0], pad]).reshape(NT, NB, BSZ)
    dstp = jnp.concatenate([edge_index[1], pad]).reshape(NT, NB, BSZ)

    h0, sd0, cm0 = _tc_proj(x, W0, a_src0, a_dst0)
    agg0, den0 = _sc_layer(h0, sd0, cm0, srcp, dstp)
    y1, h1, sd1, cm1 = _tc_epi_proj(None, agg0, den0, b0, W1, a_src1, a_dst1)
    agg1, den1 = _sc_layer(h1, sd1, cm1, srcp, dstp)
    y2, h2, sd2, cm2 = _tc_epi_proj(y1, agg1, den1, b1, W2, a_src2, a_dst2)
    agg2, den2 = _sc_layer(h2, sd2, cm2, srcp, dstp)
    return _tc_final(y2, agg2, den2, b2)


# speedup vs baseline: 1.0027x; 1.0027x over previous
import functools
import jax
import jax.numpy as jnp
from jax import lax
from jax.experimental import pallas as pl
from jax.experimental.pallas import tpu as pltpu
from jax.experimental.pallas import tpu_sc as plsc

N = 10000
E = 160000
FH = 128
NT = 16
BSZ = 32
NB = 324
EPT = NB * BSZ
E_PAD = NT * EPT
BN = 1000


def _sd_epilogue(h, as_ref, ad_ref, sd_ref, cm_ref, acc):
    s = jnp.sum(h * as_ref[...], axis=1)
    d = jnp.sum(h * ad_ref[...], axis=1)
    sd_ref[...] = jnp.concatenate(
        [s[:, None], d[:, None], jnp.zeros((s.shape[0], 6), jnp.float32)], axis=1)
    i = pl.program_id(0)
    bs, bd = jnp.max(s), jnp.max(d)

    @pl.when(i == 0)
    def _():
        acc[0] = bs
        acc[1] = bd

    @pl.when(i > 0)
    def _():
        acc[0] = jnp.maximum(acc[0], bs)
        acc[1] = jnp.maximum(acc[1], bd)

    @pl.when(i == pl.num_programs(0) - 1)
    def _():
        cm_ref[...] = jnp.full((8, 128), jnp.maximum(acc[0] + acc[1], 0.0),
                               jnp.float32)


def _tc_proj_body(x_ref, w_ref, as_ref, ad_ref, h_ref, sd_ref, cm_ref, acc):
    h = jnp.dot(x_ref[...], w_ref[...], preferred_element_type=jnp.float32)
    h_ref[0] = h[:, :FH]
    h_ref[1] = h[:, FH:]
    _sd_epilogue(h, as_ref, ad_ref, sd_ref, cm_ref, acc)


def _tc_proj(x, w, a_src, a_dst):
    fin = x.shape[1]
    return pl.pallas_call(
        _tc_proj_body,
        grid=(N // BN,),
        in_specs=[
            pl.BlockSpec((BN, fin), lambda i: (i, 0)),
            pl.BlockSpec((fin, 256), lambda i: (0, 0)),
            pl.BlockSpec((1, 256), lambda i: (0, 0)),
            pl.BlockSpec((1, 256), lambda i: (0, 0)),
        ],
        out_specs=[
            pl.BlockSpec((2, BN, FH), lambda i: (0, i, 0)),
            pl.BlockSpec((BN, 8), lambda i: (i, 0)),
            pl.BlockSpec((8, 128), lambda i: (0, 0)),
        ],
        out_shape=[
            jax.ShapeDtypeStruct((2, N, FH), jnp.float32),
            jax.ShapeDtypeStruct((N, 8), jnp.float32),
            jax.ShapeDtypeStruct((8, 128), jnp.float32),
        ],
        scratch_shapes=[pltpu.SMEM((2,), jnp.float32)],
    )(x, w, a_src[None, :], a_dst[None, :])


def _tc_epi_proj_body(yprev_ref, agg_ref, den_ref, b_ref, w_ref, as_ref,
                      ad_ref, y_ref, h_ref, sd_ref, cm_ref, acc):
    agg = jnp.concatenate([agg_ref[0], agg_ref[1]], axis=1)
    agg = agg * (1.0 / (den_ref[...] + 1e-16))
    y = jnp.maximum(agg + b_ref[...], 0.0)
    if yprev_ref is not None:
        y = y + yprev_ref[...]
    y_ref[...] = y
    h = jnp.dot(y, w_ref[...], preferred_element_type=jnp.float32)
    h_ref[0] = h[:, :FH]
    h_ref[1] = h[:, FH:]
    _sd_epilogue(h, as_ref, ad_ref, sd_ref, cm_ref, acc)


def _tc_epi_proj(yprev, agg, den, b, w, a_src, a_dst):
    if yprev is not None:
        body = _tc_epi_proj_body
        args = (yprev, agg, den, b[None, :], w, a_src[None, :], a_dst[None, :])
        prev_specs = [pl.BlockSpec((BN, 256), lambda i: (i, 0))]
    else:
        body = functools.partial(_tc_epi_proj_body, None)
        args = (agg, den, b[None, :], w, a_src[None, :], a_dst[None, :])
        prev_specs = []
    return pl.pallas_call(
        body,
        grid=(N // BN,),
        in_specs=prev_specs + [
            pl.BlockSpec((2, BN, FH), lambda i: (0, i, 0)),
            pl.BlockSpec((BN, 1), lambda i: (i, 0)),
            pl.BlockSpec((1, 256), lambda i: (0, 0)),
            pl.BlockSpec((256, 256), lambda i: (0, 0)),
            pl.BlockSpec((1, 256), lambda i: (0, 0)),
            pl.BlockSpec((1, 256), lambda i: (0, 0)),
        ],
        out_specs=[
            pl.BlockSpec((BN, 256), lambda i: (i, 0)),
            pl.BlockSpec((2, BN, FH), lambda i: (0, i, 0)),
            pl.BlockSpec((BN, 8), lambda i: (i, 0)),
            pl.BlockSpec((8, 128), lambda i: (0, 0)),
        ],
        out_shape=[
            jax.ShapeDtypeStruct((N, 256), jnp.float32),
            jax.ShapeDtypeStruct((2, N, FH), jnp.float32),
            jax.ShapeDtypeStruct((N, 8), jnp.float32),
            jax.ShapeDtypeStruct((8, 128), jnp.float32),
        ],
        scratch_shapes=[pltpu.SMEM((2,), jnp.float32)],
    )(*args)


def _tc_final_body(yprev_ref, agg_ref, den_ref, b_ref, out_ref):
    agg = jnp.concatenate([agg_ref[0], agg_ref[1]], axis=1)
    agg = agg * (1.0 / (den_ref[...] + 1e-16))
    out_ref[...] = yprev_ref[...] + jnp.maximum(agg + b_ref[...], 0.0)


def _tc_final(yprev, agg, den, b):
    return pl.pallas_call(
        _tc_final_body,
        grid=(N // BN,),
        in_specs=[
            pl.BlockSpec((BN, 256), lambda i: (i, 0)),
            pl.BlockSpec((2, BN, FH), lambda i: (0, i, 0)),
            pl.BlockSpec((BN, 1), lambda i: (i, 0)),
            pl.BlockSpec((1, 256), lambda i: (0, 0)),
        ],
        out_specs=pl.BlockSpec((BN, 256), lambda i: (i, 0)),
        out_shape=jax.ShapeDtypeStruct((N, 256), jnp.float32),
    )(yprev, agg, den, b[None, :])


def _sc_edge_body(h_hbm, s_hbm, d_hbm, c_hbm, src_hbm, dst_hbm,
                  acc_hbm, den_hbm,
                  srcb, srcoff, dstb, sv, dv, pb, rowbuf, sbuf, zb, stg, cloc,
                  s_sh, d_sh, denom_sh, acc_sh,
                  sem_r0, sem_r1, sem_r2, sem_r3, sem_r4, sem_r5,
                  sem_s0, sem_s1, sem_s2, sem_s3, sem_s4, sem_s5,
                  sem_d0, sem_d1, sem_d2, sem_a0, sem_a1, sem_a2):
    c = lax.axis_index("c")
    sid = lax.axis_index("s")
    zerov = jnp.zeros((16,), jnp.float32)
    sem_row = (sem_r0, sem_r1, sem_r2, sem_r3, sem_r4, sem_r5)
    sem_sd = (sem_s0, sem_s1, sem_s2, sem_s3, sem_s4, sem_s5)
    sem_den = (sem_d0, sem_d1, sem_d2)
    sem_acc = (sem_a0, sem_a1, sem_a2)

    pltpu.sync_copy(c_hbm, cloc)
    C = cloc[pl.ds(0, 16)][0]

    for i in range(40):
        zb[pl.ds(i * 16, 16)] = zerov

    def _zrow(g, _):
        for ii in range(16):
            for k in range(8):
                rowbuf[0, g * 16 + ii, pl.ds(k * 16, 16)] = zerov
        return ()
    lax.fori_loop(0, BSZ // 16, _zrow, ())

    rbase = sid * 632

    @pl.when(sid < 15)
    def _():
        for i in range(19):
            pltpu.sync_copy(rowbuf.at[0],
                            acc_sh.at[pl.ds(rbase + i * 32, 32)])
        pltpu.sync_copy(rowbuf.at[0, pl.ds(0, 24)],
                        acc_sh.at[pl.ds(rbase + 608, 24)])
        pltpu.sync_copy(zb.at[pl.ds(0, 632)], denom_sh.at[pl.ds(rbase, 632)])
        pltpu.sync_copy(s_hbm.at[pl.ds(rbase, 632)], stg.at[pl.ds(0, 632)])
        pltpu.sync_copy(stg.at[pl.ds(0, 632)], s_sh.at[pl.ds(rbase, 632)])
        pltpu.sync_copy(d_hbm.at[pl.ds(rbase, 632)], stg.at[pl.ds(0, 632)])
        pltpu.sync_copy(stg.at[pl.ds(0, 632)], d_sh.at[pl.ds(rbase, 632)])

    @pl.when(sid == 15)
    def _():
        for i in range(16):
            pltpu.sync_copy(rowbuf.at[0],
                            acc_sh.at[pl.ds(rbase + i * 32, 32)])
        pltpu.sync_copy(rowbuf.at[0, pl.ds(0, 8)],
                        acc_sh.at[pl.ds(rbase + 512, 8)])
        pltpu.sync_copy(zb.at[pl.ds(0, 520)], denom_sh.at[pl.ds(rbase, 520)])
        pltpu.sync_copy(s_hbm.at[pl.ds(rbase, 520)], stg.at[pl.ds(0, 520)])
        pltpu.sync_copy(stg.at[pl.ds(0, 520)], s_sh.at[pl.ds(rbase, 520)])
        pltpu.sync_copy(d_hbm.at[pl.ds(rbase, 520)], stg.at[pl.ds(0, 520)])
        pltpu.sync_copy(stg.at[pl.ds(0, 520)], d_sh.at[pl.ds(rbase, 520)])

    plsc.subcore_barrier()

    off = c * N
    ebase = sid * EPT

    def fire(m, rs, dd):
        pltpu.sync_copy(src_hbm.at[sid, m], srcb.at[rs])
        pltpu.sync_copy(dst_hbm.at[sid, m], dstb.at[dd])
        for k in range(BSZ // 16):
            srcoff[rs, pl.ds(k * 16, 16)] = (
                srcb[rs, pl.ds(k * 16, 16)] + off)
        pltpu.async_copy(h_hbm.at[srcoff.at[rs]], rowbuf.at[rs], sem_row[rs])
        pltpu.async_copy(s_sh.at[srcb.at[rs]], sv.at[rs], sem_sd[rs])
        pltpu.async_copy(d_sh.at[dstb.at[dd]], dv.at[rs], sem_sd[rs])

    def proc(m, rs, dd, ps):
        @pl.when(m >= 3)
        def _():
            pltpu.make_async_copy(
                h_hbm.at[pl.ds(0, BSZ)], sbuf.at[ps], sem_acc[ps]).wait()
            pltpu.make_async_copy(
                s_hbm.at[pl.ds(0, BSZ)], pb.at[ps], sem_den[ps]).wait()

        pltpu.make_async_copy(
            s_hbm.at[pl.ds(0, BSZ)], sv.at[rs], sem_sd[rs]).wait()
        pltpu.make_async_copy(
            s_hbm.at[pl.ds(0, BSZ)], dv.at[rs], sem_sd[rs]).wait()

        for k in range(BSZ // 16):
            e = sv[rs, pl.ds(k * 16, 16)] + dv[rs, pl.ds(k * 16, 16)]
            e = jnp.where(e >= 0.0, e, 0.2 * e)
            p = jnp.exp(e - C)
            gid = ebase + m * BSZ + k * 16 + lax.iota(jnp.int32, 16)
            pb[ps, pl.ds(k * 16, 16)] = jnp.where(gid < E, p, 0.0)
        pltpu.async_copy(pb.at[ps], denom_sh.at[dstb.at[dd]], sem_den[ps],
                         add=True)

        pltpu.make_async_copy(
            h_hbm.at[pl.ds(0, BSZ)], rowbuf.at[rs], sem_row[rs]).wait()

        def _grp(g, _):
            av = pb[ps, pl.ds(g * 16, 16)]
            for ii in range(16):
                a = av[ii]
                r = g * 16 + ii
                for k in range(8):
                    sbuf[ps, r, pl.ds(k * 16, 16)] = (
                        rowbuf[rs, r, pl.ds(k * 16, 16)] * a)
            return ()
        lax.fori_loop(0, BSZ // 16, _grp, ())
        pltpu.async_copy(sbuf.at[ps], acc_sh.at[dstb.at[dd]], sem_acc[ps],
                         add=True)

    fire(0, 0, 0)
    fire(1, 1, 1)
    fire(2, 2, 2)
    fire(3, 3, 3)

    def _outer(jo, _):
        j = 12 * jo
        for i in range(12):
            mf = j + 4 + i
            if i < 8:
                fire(mf, (4 + i) % 6, (4 + i) % 12)
            else:
                @pl.when(mf < NB)
                def _():
                    fire(mf, (4 + i) % 6, (4 + i) % 12)
            proc(j + i, i % 6, i, i % 3)
        return ()
    lax.fori_loop(0, NB // 12, _outer, ())

    for ps in range(3):
        pltpu.make_async_copy(
            h_hbm.at[pl.ds(0, BSZ)], sbuf.at[ps], sem_acc[ps]).wait()
        pltpu.make_async_copy(
            s_hbm.at[pl.ds(0, BSZ)], pb.at[ps], sem_den[ps]).wait()

    plsc.subcore_barrier()

    @pl.when(sid < 15)
    def _():
        pltpu.sync_copy(acc_sh.at[pl.ds(rbase, 632)],
                        acc_hbm.at[pl.ds(c * N + rbase, 632)])

        @pl.when(c == 0)
        def _():
            pltpu.sync_copy(denom_sh.at[pl.ds(rbase, 632)],
                            stg.at[pl.ds(0, 632)])
            pltpu.sync_copy(stg.at[pl.ds(0, 632)],
                            den_hbm.at[pl.ds(rbase, 632)])

    @pl.when(sid == 15)
    def _():
        pltpu.sync_copy(acc_sh.at[pl.ds(rbase, 520)],
                        acc_hbm.at[pl.ds(c * N + rbase, 520)])

        @pl.when(c == 0)
        def _():
            pltpu.sync_copy(denom_sh.at[pl.ds(rbase, 520)],
                            stg.at[pl.ds(0, 520)])
            pltpu.sync_copy(stg.at[pl.ds(0, 520)],
                            den_hbm.at[pl.ds(rbase, 520)])


_sc_edge = functools.partial(
    pl.kernel,
    mesh=plsc.VectorSubcoreMesh(core_axis_name="c", subcore_axis_name="s"),
    compiler_params=pltpu.CompilerParams(needs_layout_passes=False),
    out_type=[
        jax.ShapeDtypeStruct((2 * N, FH), jnp.float32),
        jax.ShapeDtypeStruct((N,), jnp.float32),
    ],
    scratch_types=[
        pltpu.VMEM((6, BSZ), jnp.int32),
        pltpu.VMEM((6, BSZ), jnp.int32),
        pltpu.VMEM((12, BSZ), jnp.int32),
        pltpu.VMEM((6, BSZ), jnp.float32),
        pltpu.VMEM((6, BSZ), jnp.float32),
        pltpu.VMEM((3, BSZ), jnp.float32),
        pltpu.VMEM((6, BSZ, FH), jnp.float32),
        pltpu.VMEM((3, BSZ, FH), jnp.float32),
        pltpu.VMEM((640,), jnp.float32),
        pltpu.VMEM((640,), jnp.float32),
        pltpu.VMEM((16,), jnp.float32),
        pltpu.VMEM_SHARED((N,), jnp.float32),
        pltpu.VMEM_SHARED((N,), jnp.float32),
        pltpu.VMEM_SHARED((N,), jnp.float32),
        pltpu.VMEM_SHARED((N, FH), jnp.float32),
        pltpu.SemaphoreType.DMA,
        pltpu.SemaphoreType.DMA,
        pltpu.SemaphoreType.DMA,
        pltpu.SemaphoreType.DMA,
        pltpu.SemaphoreType.DMA,
        pltpu.SemaphoreType.DMA,
        pltpu.SemaphoreType.DMA,
        pltpu.SemaphoreType.DMA,
        pltpu.SemaphoreType.DMA,
        pltpu.SemaphoreType.DMA,
        pltpu.SemaphoreType.DMA,
        pltpu.SemaphoreType.DMA,
        pltpu.SemaphoreType.DMA,
        pltpu.SemaphoreType.DMA,
        pltpu.SemaphoreType.DMA,
        pltpu.SemaphoreType.DMA,
        pltpu.SemaphoreType.DMA,
        pltpu.SemaphoreType.DMA,
    ],
)(_sc_edge_body)


def _sc_layer(h_split, sd, cm, srcp, dstp):
    h_flat = h_split.reshape(2 * N, FH)
    cvec = cm.reshape(-1)[:16]
    agg, den = _sc_edge(h_flat, sd[:, 0], sd[:, 1], cvec, srcp, dstp)
    return agg.reshape(2, N, FH), den.reshape(N, 1)


def kernel(x, edge_index, W0, a_src0, a_dst0, b0, W1, a_src1, a_dst1, b1,
           W2, a_src2, a_dst2, b2):
    pad = jnp.zeros((E_PAD - E,), jnp.int32)
    srcp = jnp.concatenate([edge_index[0], pad]).reshape(NT, NB, BSZ)
    dstp = jnp.concatenate([edge_index[1], pad]).reshape(NT, NB, BSZ)

    h0, sd0, cm0 = _tc_proj(x, W0, a_src0, a_dst0)
    agg0, den0 = _sc_layer(h0, sd0, cm0, srcp, dstp)
    y1, h1, sd1, cm1 = _tc_epi_proj(None, agg0, den0, b0, W1, a_src1, a_dst1)
    agg1, den1 = _sc_layer(h1, sd1, cm1, srcp, dstp)
    y2, h2, sd2, cm2 = _tc_epi_proj(y1, agg1, den1, b1, W2, a_src2, a_dst2)
    agg2, den2 = _sc_layer(h2, sd2, cm2, srcp, dstp)
    return _tc_final(y2, agg2, den2, b2)


# speedup vs baseline: 1.2576x; 1.2542x over previous
import functools
import jax
import jax.numpy as jnp
from jax import lax
from jax.experimental import pallas as pl
from jax.experimental.pallas import tpu as pltpu
from jax.experimental.pallas import tpu_sc as plsc

N = 10000
E = 160000
FH = 128
NT = 16
BSZ = 32
NB = 324
EPT = NB * BSZ
E_PAD = NT * EPT
BN = 1000


def _sd_epilogue(h, as_ref, ad_ref, sd_ref, cm_ref, acc):
    s = jnp.sum(h * as_ref[...], axis=1)
    d = jnp.sum(h * ad_ref[...], axis=1)
    sd_ref[...] = jnp.concatenate(
        [s[:, None], d[:, None], jnp.zeros((s.shape[0], 6), jnp.float32)], axis=1)
    i = pl.program_id(0)
    bs, bd = jnp.max(s), jnp.max(d)

    @pl.when(i == 0)
    def _():
        acc[0] = bs
        acc[1] = bd

    @pl.when(i > 0)
    def _():
        acc[0] = jnp.maximum(acc[0], bs)
        acc[1] = jnp.maximum(acc[1], bd)

    @pl.when(i == pl.num_programs(0) - 1)
    def _():
        cm_ref[...] = jnp.full((8, 128), jnp.maximum(acc[0] + acc[1], 0.0),
                               jnp.float32)


def _tc_proj_body(x_ref, w_ref, as_ref, ad_ref, h_ref, sd_ref, cm_ref, acc):
    h = jnp.dot(x_ref[...], w_ref[...], preferred_element_type=jnp.float32)
    h_ref[0] = h[:, :FH]
    h_ref[1] = h[:, FH:]
    _sd_epilogue(h, as_ref, ad_ref, sd_ref, cm_ref, acc)


def _tc_proj(x, w, a_src, a_dst):
    fin = x.shape[1]
    return pl.pallas_call(
        _tc_proj_body,
        grid=(N // BN,),
        in_specs=[
            pl.BlockSpec((BN, fin), lambda i: (i, 0)),
            pl.BlockSpec((fin, 256), lambda i: (0, 0)),
            pl.BlockSpec((1, 256), lambda i: (0, 0)),
            pl.BlockSpec((1, 256), lambda i: (0, 0)),
        ],
        out_specs=[
            pl.BlockSpec((2, BN, FH), lambda i: (0, i, 0)),
            pl.BlockSpec((BN, 8), lambda i: (i, 0)),
            pl.BlockSpec((8, 128), lambda i: (0, 0)),
        ],
        out_shape=[
            jax.ShapeDtypeStruct((2, N, FH), jnp.float32),
            jax.ShapeDtypeStruct((N, 8), jnp.float32),
            jax.ShapeDtypeStruct((8, 128), jnp.float32),
        ],
        scratch_shapes=[pltpu.SMEM((2,), jnp.float32)],
    )(x, w, a_src[None, :], a_dst[None, :])


def _tc_epi_proj_body(yprev_ref, agg_ref, den_ref, b_ref, w_ref, as_ref,
                      ad_ref, y_ref, h_ref, sd_ref, cm_ref, acc):
    agg = jnp.concatenate([agg_ref[0], agg_ref[1]], axis=1)
    agg = agg * (1.0 / (den_ref[...] + 1e-16))
    y = jnp.maximum(agg + b_ref[...], 0.0)
    if yprev_ref is not None:
        y = y + yprev_ref[...]
    y_ref[...] = y
    h = jnp.dot(y, w_ref[...], preferred_element_type=jnp.float32)
    h_ref[0] = h[:, :FH]
    h_ref[1] = h[:, FH:]
    _sd_epilogue(h, as_ref, ad_ref, sd_ref, cm_ref, acc)


def _tc_epi_proj(yprev, agg, den, b, w, a_src, a_dst):
    if yprev is not None:
        body = _tc_epi_proj_body
        args = (yprev, agg, den, b[None, :], w, a_src[None, :], a_dst[None, :])
        prev_specs = [pl.BlockSpec((BN, 256), lambda i: (i, 0))]
    else:
        body = functools.partial(_tc_epi_proj_body, None)
        args = (agg, den, b[None, :], w, a_src[None, :], a_dst[None, :])
        prev_specs = []
    return pl.pallas_call(
        body,
        grid=(N // BN,),
        in_specs=prev_specs + [
            pl.BlockSpec((2, BN, FH), lambda i: (0, i, 0)),
            pl.BlockSpec((BN, 1), lambda i: (i, 0)),
            pl.BlockSpec((1, 256), lambda i: (0, 0)),
            pl.BlockSpec((256, 256), lambda i: (0, 0)),
            pl.BlockSpec((1, 256), lambda i: (0, 0)),
            pl.BlockSpec((1, 256), lambda i: (0, 0)),
        ],
        out_specs=[
            pl.BlockSpec((BN, 256), lambda i: (i, 0)),
            pl.BlockSpec((2, BN, FH), lambda i: (0, i, 0)),
            pl.BlockSpec((BN, 8), lambda i: (i, 0)),
            pl.BlockSpec((8, 128), lambda i: (0, 0)),
        ],
        out_shape=[
            jax.ShapeDtypeStruct((N, 256), jnp.float32),
            jax.ShapeDtypeStruct((2, N, FH), jnp.float32),
            jax.ShapeDtypeStruct((N, 8), jnp.float32),
            jax.ShapeDtypeStruct((8, 128), jnp.float32),
        ],
        scratch_shapes=[pltpu.SMEM((2,), jnp.float32)],
    )(*args)


def _tc_final_body(yprev_ref, agg_ref, den_ref, b_ref, out_ref):
    agg = jnp.concatenate([agg_ref[0], agg_ref[1]], axis=1)
    agg = agg * (1.0 / (den_ref[...] + 1e-16))
    out_ref[...] = yprev_ref[...] + jnp.maximum(agg + b_ref[...], 0.0)


def _tc_final(yprev, agg, den, b):
    return pl.pallas_call(
        _tc_final_body,
        grid=(N // BN,),
        in_specs=[
            pl.BlockSpec((BN, 256), lambda i: (i, 0)),
            pl.BlockSpec((2, BN, FH), lambda i: (0, i, 0)),
            pl.BlockSpec((BN, 1), lambda i: (i, 0)),
            pl.BlockSpec((1, 256), lambda i: (0, 0)),
        ],
        out_specs=pl.BlockSpec((BN, 256), lambda i: (i, 0)),
        out_shape=jax.ShapeDtypeStruct((N, 256), jnp.float32),
    )(yprev, agg, den, b[None, :])


def _sc_edge_body(h_hbm, s_hbm, d_hbm, c_hbm, src_hbm, dst_hbm,
                  acc_hbm, den_hbm,
                  srcb, srcoff, dstb, sv, dv, pb, rowbuf, sbuf, zb, stg, cloc,
                  s_sh, d_sh, denom_sh, acc_sh,
                  sem_r0, sem_r1, sem_r2, sem_r3, sem_r4, sem_r5,
                  sem_s0, sem_s1, sem_s2, sem_s3, sem_s4, sem_s5,
                  sem_d0, sem_d1, sem_d2, sem_a0, sem_a1, sem_a2, sem_idx):
    c = lax.axis_index("c")
    sid = lax.axis_index("s")
    zerov = jnp.zeros((16,), jnp.float32)
    sem_row = (sem_r0, sem_r1, sem_r2, sem_r3, sem_r4, sem_r5)
    sem_sd = (sem_s0, sem_s1, sem_s2, sem_s3, sem_s4, sem_s5)
    sem_den = (sem_d0, sem_d1, sem_d2)
    sem_acc = (sem_a0, sem_a1, sem_a2)

    pltpu.sync_copy(c_hbm, cloc)
    C = cloc[pl.ds(0, 16)][0]

    for i in range(40):
        zb[pl.ds(i * 16, 16)] = zerov

    def _zrow(g, _):
        for ii in range(16):
            for k in range(8):
                rowbuf[0, g * 16 + ii, pl.ds(k * 16, 16)] = zerov
        return ()
    lax.fori_loop(0, BSZ // 16, _zrow, ())

    rbase = sid * 632

    @pl.when(sid < 15)
    def _():
        for i in range(19):
            pltpu.sync_copy(rowbuf.at[0],
                            acc_sh.at[pl.ds(rbase + i * 32, 32)])
        pltpu.sync_copy(rowbuf.at[0, pl.ds(0, 24)],
                        acc_sh.at[pl.ds(rbase + 608, 24)])
        pltpu.sync_copy(zb.at[pl.ds(0, 632)], denom_sh.at[pl.ds(rbase, 632)])
        pltpu.sync_copy(s_hbm.at[pl.ds(rbase, 632)], stg.at[pl.ds(0, 632)])
        pltpu.sync_copy(stg.at[pl.ds(0, 632)], s_sh.at[pl.ds(rbase, 632)])
        pltpu.sync_copy(d_hbm.at[pl.ds(rbase, 632)], stg.at[pl.ds(0, 632)])
        pltpu.sync_copy(stg.at[pl.ds(0, 632)], d_sh.at[pl.ds(rbase, 632)])

    @pl.when(sid == 15)
    def _():
        for i in range(16):
            pltpu.sync_copy(rowbuf.at[0],
                            acc_sh.at[pl.ds(rbase + i * 32, 32)])
        pltpu.sync_copy(rowbuf.at[0, pl.ds(0, 8)],
                        acc_sh.at[pl.ds(rbase + 512, 8)])
        pltpu.sync_copy(zb.at[pl.ds(0, 520)], denom_sh.at[pl.ds(rbase, 520)])
        pltpu.sync_copy(s_hbm.at[pl.ds(rbase, 520)], stg.at[pl.ds(0, 520)])
        pltpu.sync_copy(stg.at[pl.ds(0, 520)], s_sh.at[pl.ds(rbase, 520)])
        pltpu.sync_copy(d_hbm.at[pl.ds(rbase, 520)], stg.at[pl.ds(0, 520)])
        pltpu.sync_copy(stg.at[pl.ds(0, 520)], d_sh.at[pl.ds(rbase, 520)])

    plsc.subcore_barrier()

    off = c * N
    ebase = sid * EPT

    def fire(m, rs, bo):
        for k in range(BSZ // 16):
            srcoff[rs, pl.ds(k * 16, 16)] = (
                srcb[pl.ds(bo + k * 16, 16)] + off)
        pltpu.async_copy(h_hbm.at[srcoff.at[rs]], rowbuf.at[rs], sem_row[rs])
        pltpu.async_copy(s_sh.at[srcb.at[pl.ds(bo, BSZ)]], sv.at[rs],
                         sem_sd[rs])
        pltpu.async_copy(d_sh.at[dstb.at[pl.ds(bo, BSZ)]], dv.at[rs],
                         sem_sd[rs])

    def proc(m, rs, bo, ps):
        @pl.when(m >= 3)
        def _():
            pltpu.make_async_copy(
                h_hbm.at[pl.ds(0, BSZ)], sbuf.at[ps], sem_acc[ps]).wait()
            pltpu.make_async_copy(
                s_hbm.at[pl.ds(0, BSZ)], pb.at[ps], sem_den[ps]).wait()

        pltpu.make_async_copy(
            s_hbm.at[pl.ds(0, BSZ)], sv.at[rs], sem_sd[rs]).wait()
        pltpu.make_async_copy(
            s_hbm.at[pl.ds(0, BSZ)], dv.at[rs], sem_sd[rs]).wait()

        for k in range(BSZ // 16):
            e = sv[rs, pl.ds(k * 16, 16)] + dv[rs, pl.ds(k * 16, 16)]
            e = jnp.where(e >= 0.0, e, 0.2 * e)
            p = jnp.exp(e - C)
            gid = ebase + m * BSZ + k * 16 + lax.iota(jnp.int32, 16)
            pb[ps, pl.ds(k * 16, 16)] = jnp.where(gid < E, p, 0.0)
        pltpu.async_copy(pb.at[ps], denom_sh.at[dstb.at[pl.ds(bo, BSZ)]],
                         sem_den[ps], add=True)

        pltpu.make_async_copy(
            h_hbm.at[pl.ds(0, BSZ)], rowbuf.at[rs], sem_row[rs]).wait()

        def _grp(g, _):
            av = pb[ps, pl.ds(g * 16, 16)]
            for ii in range(16):
                a = av[ii]
                r = g * 16 + ii
                for k in range(8):
                    sbuf[ps, r, pl.ds(k * 16, 16)] = (
                        rowbuf[rs, r, pl.ds(k * 16, 16)] * a)
            return ()
        lax.fori_loop(0, BSZ // 16, _grp, ())
        pltpu.async_copy(sbuf.at[ps], acc_sh.at[dstb.at[pl.ds(bo, BSZ)]],
                         sem_acc[ps], add=True)

    GW = 12 * BSZ
    NGRP = NB // 12
    pltpu.sync_copy(src_hbm.at[sid, pl.ds(0, GW)], srcb.at[pl.ds(0, GW)])
    pltpu.sync_copy(dst_hbm.at[sid, pl.ds(0, GW)], dstb.at[pl.ds(0, GW)])
    fire(0, 0, 0 * BSZ)
    fire(1, 1, 1 * BSZ)
    fire(2, 2, 2 * BSZ)
    fire(3, 3, 3 * BSZ)

    def _outer(jo, _):
        j = 12 * jo
        bcur = (jo % 2) * GW
        bnxt = GW - bcur
        for i in range(12):
            mf = j + 4 + i
            if i < 8:
                fire(mf, (4 + i) % 6, bcur + (4 + i) * BSZ)
            else:
                if i == 8:
                    @pl.when(jo < NGRP - 1)
                    def _():
                        pltpu.make_async_copy(
                            src_hbm.at[sid, pl.ds(0, GW)],
                            srcb.at[pl.ds(0, GW)], sem_idx).wait()
                        pltpu.make_async_copy(
                            src_hbm.at[sid, pl.ds(0, GW)],
                            dstb.at[pl.ds(0, GW)], sem_idx).wait()

                @pl.when(mf < NB)
                def _():
                    fire(mf, (4 + i) % 6, bnxt + (i - 8) * BSZ)
            proc(j + i, i % 6, bcur + i * BSZ, i % 3)
            if i == 2:
                @pl.when(jo < NGRP - 1)
                def _():
                    pltpu.async_copy(
                        src_hbm.at[sid, pl.ds((j + 12) * BSZ, GW)],
                        srcb.at[pl.ds(bnxt, GW)], sem_idx)
                    pltpu.async_copy(
                        dst_hbm.at[sid, pl.ds((j + 12) * BSZ, GW)],
                        dstb.at[pl.ds(bnxt, GW)], sem_idx)
        return ()
    lax.fori_loop(0, NGRP, _outer, ())

    for ps in range(3):
        pltpu.make_async_copy(
            h_hbm.at[pl.ds(0, BSZ)], sbuf.at[ps], sem_acc[ps]).wait()
        pltpu.make_async_copy(
            s_hbm.at[pl.ds(0, BSZ)], pb.at[ps], sem_den[ps]).wait()

    plsc.subcore_barrier()

    @pl.when(sid < 15)
    def _():
        pltpu.sync_copy(acc_sh.at[pl.ds(rbase, 632)],
                        acc_hbm.at[pl.ds(c * N + rbase, 632)])

        @pl.when(c == 0)
        def _():
            pltpu.sync_copy(denom_sh.at[pl.ds(rbase, 632)],
                            stg.at[pl.ds(0, 632)])
            pltpu.sync_copy(stg.at[pl.ds(0, 632)],
                            den_hbm.at[pl.ds(rbase, 632)])

    @pl.when(sid == 15)
    def _():
        pltpu.sync_copy(acc_sh.at[pl.ds(rbase, 520)],
                        acc_hbm.at[pl.ds(c * N + rbase, 520)])

        @pl.when(c == 0)
        def _():
            pltpu.sync_copy(denom_sh.at[pl.ds(rbase, 520)],
                            stg.at[pl.ds(0, 520)])
            pltpu.sync_copy(stg.at[pl.ds(0, 520)],
                            den_hbm.at[pl.ds(rbase, 520)])


_sc_edge = functools.partial(
    pl.kernel,
    mesh=plsc.VectorSubcoreMesh(core_axis_name="c", subcore_axis_name="s"),
    compiler_params=pltpu.CompilerParams(needs_layout_passes=False),
    out_type=[
        jax.ShapeDtypeStruct((2 * N, FH), jnp.float32),
        jax.ShapeDtypeStruct((N,), jnp.float32),
    ],
    scratch_types=[
        pltpu.VMEM((24 * BSZ,), jnp.int32),
        pltpu.VMEM((6, BSZ), jnp.int32),
        pltpu.VMEM((24 * BSZ,), jnp.int32),
        pltpu.VMEM((6, BSZ), jnp.float32),
        pltpu.VMEM((6, BSZ), jnp.float32),
        pltpu.VMEM((3, BSZ), jnp.float32),
        pltpu.VMEM((6, BSZ, FH), jnp.float32),
        pltpu.VMEM((3, BSZ, FH), jnp.float32),
        pltpu.VMEM((640,), jnp.float32),
        pltpu.VMEM((640,), jnp.float32),
        pltpu.VMEM((16,), jnp.float32),
        pltpu.VMEM_SHARED((N,), jnp.float32),
        pltpu.VMEM_SHARED((N,), jnp.float32),
        pltpu.VMEM_SHARED((N,), jnp.float32),
        pltpu.VMEM_SHARED((N, FH), jnp.float32),
        pltpu.SemaphoreType.DMA,
        pltpu.SemaphoreType.DMA,
        pltpu.SemaphoreType.DMA,
        pltpu.SemaphoreType.DMA,
        pltpu.SemaphoreType.DMA,
        pltpu.SemaphoreType.DMA,
        pltpu.SemaphoreType.DMA,
        pltpu.SemaphoreType.DMA,
        pltpu.SemaphoreType.DMA,
        pltpu.SemaphoreType.DMA,
        pltpu.SemaphoreType.DMA,
        pltpu.SemaphoreType.DMA,
        pltpu.SemaphoreType.DMA,
        pltpu.SemaphoreType.DMA,
        pltpu.SemaphoreType.DMA,
        pltpu.SemaphoreType.DMA,
        pltpu.SemaphoreType.DMA,
        pltpu.SemaphoreType.DMA,
        pltpu.SemaphoreType.DMA,
    ],
)(_sc_edge_body)


def _sc_layer(h_split, sd, cm, srcp, dstp):
    h_flat = h_split.reshape(2 * N, FH)
    cvec = cm.reshape(-1)[:16]
    agg, den = _sc_edge(h_flat, sd[:, 0], sd[:, 1], cvec, srcp, dstp)
    return agg.reshape(2, N, FH), den.reshape(N, 1)


def kernel(x, edge_index, W0, a_src0, a_dst0, b0, W1, a_src1, a_dst1, b1,
           W2, a_src2, a_dst2, b2):
    pad = jnp.zeros((E_PAD - E,), jnp.int32)
    srcp = jnp.concatenate([edge_index[0], pad]).reshape(NT, EPT)
    dstp = jnp.concatenate([edge_index[1], pad]).reshape(NT, EPT)

    h0, sd0, cm0 = _tc_proj(x, W0, a_src0, a_dst0)
    agg0, den0 = _sc_layer(h0, sd0, cm0, srcp, dstp)
    y1, h1, sd1, cm1 = _tc_epi_proj(None, agg0, den0, b0, W1, a_src1, a_dst1)
    agg1, den1 = _sc_layer(h1, sd1, cm1, srcp, dstp)
    y2, h2, sd2, cm2 = _tc_epi_proj(y1, agg1, den1, b1, W2, a_src2, a_dst2)
    agg2, den2 = _sc_layer(h2, sd2, cm2, srcp, dstp)
    return _tc_final(y2, agg2, den2, b2)
